# baseline (speedup 1.0000x reference)
import numpy as np
import jax
import jax.numpy as jnp
from jax.experimental import pallas as pl
from jax.experimental.pallas import tpu as pltpu

_N, _L, _K = 128, 8192, 20
_C = 2560
_R = _L // 128
_G = _C // _K


def _threefry_bits(x1):
    k0 = jnp.uint32(0)
    k1 = jnp.uint32(42)
    k2 = jnp.uint32(0 ^ 42 ^ 0x1BD11BDA)
    ks = (k0, k1, k2)
    rot = ((13, 15, 26, 6), (17, 29, 16, 24))
    x0 = jnp.zeros_like(x1)
    x1 = x1 + k1
    for i in range(5):
        for r in rot[i % 2]:
            x0 = x0 + x1
            x1 = (x1 << r) | (x1 >> (32 - r))
            x1 = x1 ^ x0
        x0 = x0 + ks[(i + 1) % 3]
        x1 = x1 + ks[(i + 2) % 3] + jnp.uint32(i + 1)
    return x0 ^ x1


def _body(dt_ref, ct_ref, vc_ref, xt_ref, mk_ref, x_out, c_out):
    n = pl.program_id(0)
    ct = ct_ref[0]
    vc = vc_ref[0]
    dtn = dt_ref[n]
    c_new = ct + vc * dtn
    probs = jnp.clip(c_new, 0.0, 1.0) + 1e-8
    v = jnp.log(probs)

    row = jax.lax.broadcasted_iota(jnp.int32, (_R, _C), 0)
    lane = jax.lax.broadcasted_iota(jnp.int32, (_R, _C), 1)
    flat = (n * _R + row) * _C + lane
    bits = _threefry_bits(flat.astype(jnp.uint32))
    fb = (bits >> 9) | jnp.uint32(0x3F800000)
    floats = jax.lax.bitcast_convert_type(fb, jnp.float32) - 1.0
    tiny = jnp.float32(np.finfo(np.float32).tiny)
    u = jnp.maximum(tiny, floats + tiny)
    v = v + (-jnp.log(-jnp.log(u)))

    pg = lane % _K
    cur_v = v
    cur_i = pg.astype(jnp.float32)
    neg_inf = jnp.float32(-np.inf)
    for s in (1, 2, 4, 8, 16):
        cand_v = pltpu.roll(cur_v, _C - s, 1)
        cand_i = pltpu.roll(cur_i, _C - s, 1)
        cand_v = jnp.where(pg + s < _K, cand_v, neg_inf)
        take = cand_v > cur_v
        cur_v = jnp.where(take, cand_v, cur_v)
        cur_i = jnp.where(take, cand_i, cur_i)

    crow = jax.lax.broadcasted_iota(jnp.int32, (_C, _G), 0)
    ccol = jax.lax.broadcasted_iota(jnp.int32, (_C, _G), 1)
    e1 = (crow // _K == ccol).astype(jnp.float32)
    contrib = jnp.where(pg == 0, cur_i, 0.0)
    xs = jnp.dot(contrib, e1, preferred_element_type=jnp.float32)

    mk = mk_ref[0]
    xt = xt_ref[0]
    x_out[0] = jnp.where(mk != 0, xs.astype(jnp.int32), xt)

    trow = jax.lax.broadcasted_iota(jnp.int32, (_G, _C), 0)
    tcol = jax.lax.broadcasted_iota(jnp.int32, (_G, _C), 1)
    e1t = (tcol // _K == trow).astype(jnp.float32)
    mke = jnp.dot(mk.astype(jnp.float32), e1t,
                  preferred_element_type=jnp.float32)
    c_out[0] = jnp.where(mke > 0.5, c_new, ct)


def kernel(xt, ct, vc_t, dt, mask):
    ct3 = ct.reshape(_N, _R, _C)
    vc3 = vc_t.reshape(_N, _R, _C)
    xt3 = xt.reshape(_N, _R, 128)
    mk3 = mask.astype(jnp.int32).reshape(_N, _R, 128)
    x_new, c_new = pl.pallas_call(
        _body,
        grid=(_N,),
        in_specs=[
            pl.BlockSpec(memory_space=pltpu.SMEM),
            pl.BlockSpec((1, _R, _C), lambda n: (n, 0, 0)),
            pl.BlockSpec((1, _R, _C), lambda n: (n, 0, 0)),
            pl.BlockSpec((1, _R, 128), lambda n: (n, 0, 0)),
            pl.BlockSpec((1, _R, 128), lambda n: (n, 0, 0)),
        ],
        out_specs=[
            pl.BlockSpec((1, _R, 128), lambda n: (n, 0, 0)),
            pl.BlockSpec((1, _R, _C), lambda n: (n, 0, 0)),
        ],
        out_shape=[
            jax.ShapeDtypeStruct((_N, _R, 128), jnp.int32),
            jax.ShapeDtypeStruct((_N, _R, _C), jnp.float32),
        ],
    )(dt, ct3, vc3, xt3, mk3)
    return x_new.reshape(_N, _L), c_new.reshape(_N, _L, _K)
